# baseline (speedup 1.0000x reference)
import functools

import jax
import jax.numpy as jnp
from jax import lax
from jax.experimental import pallas as pl
from jax.experimental.pallas import tpu as pltpu
from jax.experimental.pallas import tpu_sc as plsc

NUM_USERS = 1000000
EMBED = 128
FEAT_ITEM = 256
B = 16384

NC = 2
NS = 16
NW = NC * NS
B_PER_W = B // NW
CHUNK = 128
N_CHUNKS = B_PER_W // CHUNK

_sc_mesh = plsc.VectorSubcoreMesh(core_axis_name="c", subcore_axis_name="s")


@functools.partial(
    pl.kernel,
    out_type=jax.ShapeDtypeStruct((B, EMBED), jnp.float32),
    mesh=_sc_mesh,
    scratch_types=[
        pltpu.VMEM((N_CHUNKS, CHUNK), jnp.int32),
        pltpu.VMEM((B_PER_W, EMBED), jnp.float32),
        pltpu.SemaphoreType.DMA,
    ],
)
def _sc_gather(table_hbm, idx_hbm, out_hbm, idx_v, rows_v, sem):
    wid = lax.axis_index("s") * NC + lax.axis_index("c")
    base = wid * B_PER_W
    pltpu.sync_copy(idx_hbm.at[wid], idx_v)
    copies = []
    for j in range(N_CHUNKS):
        copies.append(
            pltpu.async_copy(
                table_hbm.at[idx_v.at[j]],
                rows_v.at[pl.ds(j * CHUNK, CHUNK)],
                sem,
            )
        )
    for c in copies:
        c.wait()
    pltpu.sync_copy(rows_v, out_hbm.at[pl.ds(base, B_PER_W)])


def _mm_body(x_ref, w_ref, o_ref):
    o_ref[...] = jnp.dot(x_ref[...], w_ref[...],
                         preferred_element_type=jnp.float32)


_MM_BLOCK = 2048


def _item_proj(input_item, proj_item):
    grid = (B // _MM_BLOCK,)
    return pl.pallas_call(
        _mm_body,
        grid=grid,
        in_specs=[
            pl.BlockSpec((_MM_BLOCK, FEAT_ITEM), lambda i: (i, 0)),
            pl.BlockSpec((FEAT_ITEM, EMBED), lambda i: (0, 0)),
        ],
        out_specs=pl.BlockSpec((_MM_BLOCK, EMBED), lambda i: (i, 0)),
        out_shape=jax.ShapeDtypeStruct((B, EMBED), jnp.float32),
    )(input_item, proj_item)


def kernel(input_item, input_nodes_user, user_table, proj_item):
    idx3 = input_nodes_user.reshape(NW, N_CHUNKS, CHUNK)
    emb_user = _sc_gather(user_table, idx3)
    return (emb_user, emb_user)


# speedup vs baseline: 1.0183x; 1.0183x over previous
import functools

import jax
import jax.numpy as jnp
from jax import lax
from jax.experimental import pallas as pl
from jax.experimental.pallas import tpu as pltpu
from jax.experimental.pallas import tpu_sc as plsc

NUM_USERS = 1000000
EMBED = 128
FEAT_ITEM = 256
B = 16384

NC = 2
NS = 16
NW = NC * NS
B_PER_W = B // NW
CHUNK = 128
N_CHUNKS = B_PER_W // CHUNK

_sc_mesh = plsc.VectorSubcoreMesh(core_axis_name="c", subcore_axis_name="s")


@functools.partial(
    pl.kernel,
    out_type=jax.ShapeDtypeStruct((B, EMBED), jnp.float32),
    mesh=_sc_mesh,
    scratch_types=[
        pltpu.VMEM((N_CHUNKS, CHUNK), jnp.int32),
        pltpu.VMEM((B_PER_W, EMBED), jnp.float32),
    ]
    + [pltpu.SemaphoreType.DMA] * N_CHUNKS
    + [pltpu.SemaphoreType.DMA],
)
def _sc_gather(table_hbm, idx_hbm, out_hbm, idx_v, rows_v, s0, s1, s2, s3,
               wsem):
    gsems = (s0, s1, s2, s3)
    wid = lax.axis_index("s") * NC + lax.axis_index("c")
    base = wid * B_PER_W
    pltpu.sync_copy(idx_hbm.at[wid], idx_v)
    gathers = [
        pltpu.async_copy(
            table_hbm.at[idx_v.at[j]],
            rows_v.at[pl.ds(j * CHUNK, CHUNK)],
            gsems[j],
        )
        for j in range(N_CHUNKS)
    ]
    writes = []
    for j in range(N_CHUNKS):
        gathers[j].wait()
        writes.append(
            pltpu.async_copy(
                rows_v.at[pl.ds(j * CHUNK, CHUNK)],
                out_hbm.at[pl.ds(base + j * CHUNK, CHUNK)],
                wsem,
            )
        )
    for c in writes:
        c.wait()


def _mm_body(x_ref, w_ref, o_ref):
    o_ref[...] = jnp.dot(x_ref[...], w_ref[...],
                         preferred_element_type=jnp.float32)


_MM_BLOCK = 2048


def _item_proj(input_item, proj_item):
    grid = (B // _MM_BLOCK,)
    return pl.pallas_call(
        _mm_body,
        grid=grid,
        in_specs=[
            pl.BlockSpec((_MM_BLOCK, FEAT_ITEM), lambda i: (i, 0)),
            pl.BlockSpec((FEAT_ITEM, EMBED), lambda i: (0, 0)),
        ],
        out_specs=pl.BlockSpec((_MM_BLOCK, EMBED), lambda i: (i, 0)),
        out_shape=jax.ShapeDtypeStruct((B, EMBED), jnp.float32),
    )(input_item, proj_item)


def kernel(input_item, input_nodes_user, user_table, proj_item):
    idx3 = input_nodes_user.reshape(NW, N_CHUNKS, CHUNK)
    emb_user = _sc_gather(user_table, idx3)
    emb_item = _item_proj(input_item, proj_item)
    return (emb_user, emb_item)


# speedup vs baseline: 1.0379x; 1.0193x over previous
import functools

import jax
import jax.numpy as jnp
from jax import lax
from jax.experimental import pallas as pl
from jax.experimental.pallas import tpu as pltpu
from jax.experimental.pallas import tpu_sc as plsc

NUM_USERS = 1000000
EMBED = 128
FEAT_ITEM = 256
B = 16384

NC = 2
NS = 16
NW = NC * NS
B_PER_W = B // NW
CHUNK = 128
N_CHUNKS = B_PER_W // CHUNK

_sc_mesh = plsc.VectorSubcoreMesh(core_axis_name="c", subcore_axis_name="s")


@functools.partial(
    pl.kernel,
    out_type=jax.ShapeDtypeStruct((B, EMBED), jnp.float32),
    mesh=_sc_mesh,
    scratch_types=[
        pltpu.VMEM((N_CHUNKS, CHUNK), jnp.int32),
        pltpu.VMEM((B_PER_W, EMBED), jnp.float32),
    ]
    + [pltpu.SemaphoreType.DMA] * N_CHUNKS
    + [pltpu.SemaphoreType.DMA],
)
def _sc_gather(table_hbm, idx_hbm, out_hbm, idx_v, rows_v, s0, s1, s2, s3,
               wsem):
    gsems = (s0, s1, s2, s3)
    wid = lax.axis_index("s") * NC + lax.axis_index("c")
    base = wid * B_PER_W
    pltpu.sync_copy(idx_hbm.at[wid], idx_v)
    gathers = [
        pltpu.async_copy(
            table_hbm.at[idx_v.at[j]],
            rows_v.at[pl.ds(j * CHUNK, CHUNK)],
            gsems[j],
        )
        for j in range(N_CHUNKS)
    ]
    writes = []
    for j in range(N_CHUNKS):
        gathers[j].wait()
        writes.append(
            pltpu.async_copy(
                rows_v.at[pl.ds(j * CHUNK, CHUNK)],
                out_hbm.at[pl.ds(base + j * CHUNK, CHUNK)],
                wsem,
            )
        )
    for c in writes:
        c.wait()


_MM_SUB = 1024
_MM_PAR = 4
_MM_STEP = _MM_SUB * _MM_PAR


def _mm_body(x0, x1, x2, x3, w_ref, o_ref):
    for s, xr in enumerate((x0, x1, x2, x3)):
        o_ref[pl.ds(s * _MM_SUB, _MM_SUB), :] = jnp.dot(
            xr[...], w_ref[...], preferred_element_type=jnp.float32)


def _item_proj(input_item, proj_item):
    grid = (B // _MM_STEP,)
    in_specs = [
        pl.BlockSpec((_MM_SUB, FEAT_ITEM),
                     lambda i, s=s: (_MM_PAR * i + s, 0))
        for s in range(_MM_PAR)
    ] + [pl.BlockSpec((FEAT_ITEM, EMBED), lambda i: (0, 0))]
    return pl.pallas_call(
        _mm_body,
        grid=grid,
        in_specs=in_specs,
        out_specs=pl.BlockSpec((_MM_STEP, EMBED), lambda i: (i, 0)),
        out_shape=jax.ShapeDtypeStruct((B, EMBED), jnp.float32),
    )(input_item, input_item, input_item, input_item, proj_item)


def kernel(input_item, input_nodes_user, user_table, proj_item):
    idx3 = input_nodes_user.reshape(NW, N_CHUNKS, CHUNK)
    emb_user = _sc_gather(user_table, idx3)
    emb_item = _item_proj(input_item, proj_item)
    return (emb_user, emb_item)
